# Initial kernel scaffold; baseline (speedup 1.0000x reference)
import jax
import jax.numpy as jnp
from jax.experimental import pallas as pl


def kernel(state_particles, param_particles, log_weights):
    raise NotImplementedError("write your pallas kernel here")


# speedup vs baseline: 4.6332x; 4.6332x over previous
import functools

import jax
import jax.numpy as jnp
from jax import lax
from jax.experimental import pallas as pl
from jax.experimental.pallas import tpu as pltpu
from jax.experimental.pallas import tpu_sc as plsc

_B, _K, _S, _P = 1024, 1024, 32, 16
_NC, _NS = 2, 16
_NWORK = _NC * _NS
_RPW = _B // _NWORK
_L = 16
_NCH = _K // _L
_EPS = 1e-10


def _prep_body(lw_ref, c_ref):
    lw = lw_ref[...]
    w = jax.nn.softmax(lw, axis=-1)
    prop = 0.5 * w + jnp.float32(0.5 / _K)
    rows = prop.shape[0]
    lane = lax.broadcasted_iota(jnp.int32, (rows, _K), 1)
    inblk = lane & 127
    c = prop
    for i in range(1, 128):
        shifted = jnp.concatenate(
            [jnp.zeros((rows, 1), jnp.float32), c[:, : _K - 1]], axis=1)
        c = jnp.where(inblk == i, c + shifted, c)
    off = jnp.zeros((rows, _K), jnp.float32)
    running = jnp.zeros((rows, 1), jnp.float32)
    for b in range(1, _K // 128):
        s_prev = jnp.sum(
            jnp.where(lane == b * 128 - 1, c, 0.0), axis=-1, keepdims=True)
        running = running + s_prev
        off = jnp.where(lane >= b * 128, jnp.broadcast_to(running, off.shape),
                        off)
    c_ref[...] = c + off


def _prep(log_weights):
    blk = 128
    return pl.pallas_call(
        _prep_body,
        grid=(_B // blk,),
        in_specs=[pl.BlockSpec((blk, _K), lambda i: (i, 0))],
        out_specs=pl.BlockSpec((blk, _K), lambda i: (i, 0)),
        out_shape=jax.ShapeDtypeStruct((_B, _K), jnp.float32),
    )(log_weights)


def _fin_body(nw_ref, out_ref):
    l = jnp.log(nw_ref[...] + _EPS)
    mx = jnp.max(l, axis=-1, keepdims=True)
    lse = mx + jnp.log(jnp.sum(jnp.exp(l - mx), axis=-1, keepdims=True))
    out_ref[...] = l - lse


def _finish(nw):
    blk = 128
    return pl.pallas_call(
        _fin_body,
        grid=(_B // blk,),
        in_specs=[pl.BlockSpec((blk, _K), lambda i: (i, 0))],
        out_specs=pl.BlockSpec((blk, _K), lambda i: (i, 0)),
        out_shape=jax.ShapeDtypeStruct((_B, _K), jnp.float32),
    )(nw)


def _splat(x):
    return jnp.full((_L,), x, jnp.int32)


def _sc_body(c_hbm, p_hbm, state_hbm, param_hbm,
             out_s, out_p, out_nw,
             c_blk, p_blk, nw_blk, hist, idx_s,
             sslab, pslab, oslab_s, oslab_p):
    wid = lax.axis_index("s") * _NC + lax.axis_index("c")

    def tr_body(t, _):
        r0 = (wid * 4 + t) * 8
        pltpu.sync_copy(c_hbm.at[pl.ds(r0, 8)], c_blk)
        pltpu.sync_copy(p_hbm.at[pl.ds(r0, 8)], p_blk)

        def row_body(s, _):
            b = r0 + s
            pltpu.sync_copy(state_hbm.at[b], sslab)
            pltpu.sync_copy(param_hbm.at[b], pslab)

            def zero_body(i, _):
                hist[pl.ds(i * _L, _L)] = jnp.zeros((_L,), jnp.int32)
                return 0

            lax.fori_loop(0, _NCH, zero_body, 0)

            def m_body(j, _):
                c16 = c_blk[s, pl.ds(j * _L, _L)]
                t16 = c16 * jnp.float32(_K)
                i0 = t16.astype(jnp.int32)
                base = jnp.maximum(i0 - 1, 0)
                m = base
                for d in range(3):
                    cand = base + d
                    valid = cand < _K
                    pv = plsc.load_gather(
                        p_blk, [_splat(s), jnp.minimum(cand, _K - 1)],
                        mask=valid)
                    m = m + (valid & (pv <= c16)).astype(jnp.int32)
                plsc.addupdate_scatter(
                    hist, [jnp.minimum(m, _K - 1)],
                    jnp.full((_L,), 1, jnp.int32), mask=m < _K)
                return 0

            lax.fori_loop(0, _NCH, m_body, 0)

            def cs_body(j, carry):
                h16 = hist[pl.ds(j * _L, _L)]
                cs = plsc.cumsum(h16) + carry
                idx16 = jnp.minimum(cs, _K - 1)
                idx_s[pl.ds(j * _L, _L)] = idx16
                cg = plsc.load_gather(c_blk, [_splat(s), idx16])
                cgm = plsc.load_gather(
                    c_blk, [_splat(s), jnp.maximum(idx16 - 1, 0)])
                prop_g = cg - jnp.where(idx16 > 0, cgm, 0.0)
                w_g = 2.0 * prop_g - jnp.float32(1.0 / _K)
                nw_blk[s, pl.ds(j * _L, _L)] = w_g / (prop_g + _EPS)
                return jnp.max(cs)

            lax.fori_loop(0, _NCH, cs_body, jnp.int32(0))

            def g_body(j, _):
                sl = pl.ds(j * _L, _L)
                idx16 = idx_s[sl]
                for ss in range(_S):
                    oslab_s[ss, sl] = plsc.load_gather(
                        sslab, [_splat(ss), idx16])
                for ss in range(_P):
                    oslab_p[ss, sl] = plsc.load_gather(
                        pslab, [_splat(ss), idx16])
                return 0

            lax.fori_loop(0, _NCH, g_body, 0)

            pltpu.sync_copy(oslab_s, out_s.at[b])
            pltpu.sync_copy(oslab_p, out_p.at[b])
            return 0

        lax.fori_loop(0, 8, row_body, 0)
        pltpu.sync_copy(nw_blk, out_nw.at[pl.ds(r0, 8)])
        return 0

    lax.fori_loop(0, _RPW // 8, tr_body, 0)


def _sc_resample(c, p, state_t, param_t):
    mesh = plsc.VectorSubcoreMesh(core_axis_name="c", subcore_axis_name="s")
    f = pl.kernel(
        _sc_body,
        out_type=(
            jax.ShapeDtypeStruct((_B, _S, _K), jnp.float32),
            jax.ShapeDtypeStruct((_B, _P, _K), jnp.float32),
            jax.ShapeDtypeStruct((_B, _K), jnp.float32),
        ),
        mesh=mesh,
        compiler_params=pltpu.CompilerParams(
            needs_layout_passes=False, use_tc_tiling_on_sc=True),
        scratch_types=[
            pltpu.VMEM((8, _K), jnp.float32),
            pltpu.VMEM((8, _K), jnp.float32),
            pltpu.VMEM((8, _K), jnp.float32),
            pltpu.VMEM((_K,), jnp.int32),
            pltpu.VMEM((_K,), jnp.int32),
            pltpu.VMEM((_S, _K), jnp.float32),
            pltpu.VMEM((_P, _K), jnp.float32),
            pltpu.VMEM((_S, _K), jnp.float32),
            pltpu.VMEM((_P, _K), jnp.float32),
        ],
    )
    return f(c, p, state_t, param_t)


def kernel(state_particles, param_particles, log_weights):
    u = jax.random.uniform(jax.random.key(42), (_B, _K), dtype=jnp.float32)
    positions = (jnp.arange(_K, dtype=jnp.float32)[None, :] + u) / _K

    c = _prep(log_weights)
    new_s, new_p, nw = _sc_resample(
        c, positions,
        jnp.transpose(state_particles, (0, 2, 1)),
        jnp.transpose(param_particles, (0, 2, 1)),
    )
    new_log_weights = _finish(nw)
    return (jnp.transpose(new_s, (0, 2, 1)), jnp.transpose(new_p, (0, 2, 1)),
            new_log_weights)


# speedup vs baseline: 5.7292x; 1.2365x over previous
import functools

import jax
import jax.numpy as jnp
from jax import lax
from jax.experimental import pallas as pl
from jax.experimental.pallas import tpu as pltpu
from jax.experimental.pallas import tpu_sc as plsc

_B, _K, _S, _P = 1024, 1024, 32, 16
_NC, _NS = 2, 16
_NWORK = _NC * _NS
_RPW = _B // _NWORK
_L = 16
_NCH = _K // _L
_EPS = 1e-10


def _prep_body(lw_ref, c_ref):
    lw = lw_ref[...]
    w = jax.nn.softmax(lw, axis=-1)
    prop = 0.5 * w + jnp.float32(0.5 / _K)
    rows = prop.shape[0]
    lane = lax.broadcasted_iota(jnp.int32, (rows, _K), 1)
    inblk = lane & 127
    c = prop
    for i in range(1, 128):
        shifted = jnp.concatenate(
            [jnp.zeros((rows, 1), jnp.float32), c[:, : _K - 1]], axis=1)
        c = jnp.where(inblk == i, c + shifted, c)
    off = jnp.zeros((rows, _K), jnp.float32)
    running = jnp.zeros((rows, 1), jnp.float32)
    for b in range(1, _K // 128):
        s_prev = jnp.sum(
            jnp.where(lane == b * 128 - 1, c, 0.0), axis=-1, keepdims=True)
        running = running + s_prev
        off = jnp.where(lane >= b * 128, jnp.broadcast_to(running, off.shape),
                        off)
    c_ref[...] = c + off


def _prep(log_weights):
    blk = 128
    return pl.pallas_call(
        _prep_body,
        grid=(_B // blk,),
        in_specs=[pl.BlockSpec((blk, _K), lambda i: (i, 0))],
        out_specs=pl.BlockSpec((blk, _K), lambda i: (i, 0)),
        out_shape=jax.ShapeDtypeStruct((_B, _K), jnp.float32),
    )(log_weights)


def _fin_body(nw_ref, out_ref):
    l = jnp.log(nw_ref[...] + _EPS)
    mx = jnp.max(l, axis=-1, keepdims=True)
    lse = mx + jnp.log(jnp.sum(jnp.exp(l - mx), axis=-1, keepdims=True))
    out_ref[...] = l - lse


def _finish(nw):
    blk = 128
    return pl.pallas_call(
        _fin_body,
        grid=(_B // blk,),
        in_specs=[pl.BlockSpec((blk, _K), lambda i: (i, 0))],
        out_specs=pl.BlockSpec((blk, _K), lambda i: (i, 0)),
        out_shape=jax.ShapeDtypeStruct((_B, _K), jnp.float32),
    )(nw)


def _splat(x):
    return jnp.full((_L,), x, jnp.int32)


def _sc_body(c_hbm, p_hbm, state_hbm, param_hbm,
             out_s, out_p, out_nw,
             c_blk, p_blk, nw_blk, hist, idx_s,
             sslab, pslab, oslab_s, oslab_p,
             sem_in, sem_out):
    wid = lax.axis_index("s") * _NC + lax.axis_index("c")

    def tr_body(t, _):
        r0 = (wid * 4 + t) * 8
        pltpu.sync_copy(c_hbm.at[pl.ds(r0, 8)], c_blk)
        pltpu.sync_copy(p_hbm.at[pl.ds(r0, 8)], p_blk)

        def row_body(s, _):
            b = r0 + s
            in_s = pltpu.async_copy(state_hbm.at[b], sslab, sem_in)
            in_p = pltpu.async_copy(param_hbm.at[b], pslab, sem_in)

            def zero_body(i, _):
                hist[pl.ds(i * _L, _L)] = jnp.zeros((_L,), jnp.int32)
                return 0

            lax.fori_loop(0, _NCH, zero_body, 0)

            def m_body(j, _):
                c16 = c_blk[s, pl.ds(j * _L, _L)]
                t16 = c16 * jnp.float32(_K)
                i0 = t16.astype(jnp.int32)
                base = jnp.maximum(i0 - 1, 0)
                m = base
                for d in range(3):
                    cand = base + d
                    valid = cand < _K
                    pv = plsc.load_gather(
                        p_blk, [_splat(s), jnp.minimum(cand, _K - 1)],
                        mask=valid)
                    m = m + (valid & (pv <= c16)).astype(jnp.int32)
                plsc.addupdate_scatter(
                    hist, [jnp.minimum(m, _K - 1)],
                    jnp.full((_L,), 1, jnp.int32), mask=m < _K)
                return 0

            lax.fori_loop(0, _NCH, m_body, 0)

            def cs_body(j, carry):
                h16 = hist[pl.ds(j * _L, _L)]
                cs = plsc.cumsum(h16) + carry
                idx16 = jnp.minimum(cs, _K - 1)
                idx_s[pl.ds(j * _L, _L)] = idx16
                cg = plsc.load_gather(c_blk, [_splat(s), idx16])
                cgm = plsc.load_gather(
                    c_blk, [_splat(s), jnp.maximum(idx16 - 1, 0)])
                prop_g = cg - jnp.where(idx16 > 0, cgm, 0.0)
                w_g = 2.0 * prop_g - jnp.float32(1.0 / _K)
                nw_blk[s, pl.ds(j * _L, _L)] = w_g / (prop_g + _EPS)
                return jnp.max(cs)

            lax.fori_loop(0, _NCH, cs_body, jnp.int32(0))

            in_s.wait()
            in_p.wait()
            @pl.when(t * 8 + s > 0)
            def _():
                pltpu.make_async_copy(state_hbm.at[b], oslab_s, sem_out).wait()
                pltpu.make_async_copy(param_hbm.at[b], oslab_p, sem_out).wait()

            def g_body(j, _):
                sl = pl.ds(j * _L, _L)
                idx16 = idx_s[sl]
                for ss in range(_S):
                    oslab_s[ss, sl] = plsc.load_gather(
                        sslab, [_splat(ss), idx16])
                for ss in range(_P):
                    oslab_p[ss, sl] = plsc.load_gather(
                        pslab, [_splat(ss), idx16])
                return 0

            lax.fori_loop(0, _NCH, g_body, 0)

            pltpu.async_copy(oslab_s, out_s.at[b], sem_out)
            pltpu.async_copy(oslab_p, out_p.at[b], sem_out)
            return 0

        lax.fori_loop(0, 8, row_body, 0)
        pltpu.sync_copy(nw_blk, out_nw.at[pl.ds(r0, 8)])
        return 0

    lax.fori_loop(0, _RPW // 8, tr_body, 0)
    pltpu.make_async_copy(state_hbm.at[0], oslab_s, sem_out).wait()
    pltpu.make_async_copy(param_hbm.at[0], oslab_p, sem_out).wait()


def _sc_resample(c, p, state_t, param_t):
    mesh = plsc.VectorSubcoreMesh(core_axis_name="c", subcore_axis_name="s")
    f = pl.kernel(
        _sc_body,
        out_type=(
            jax.ShapeDtypeStruct((_B, _S, _K), jnp.float32),
            jax.ShapeDtypeStruct((_B, _P, _K), jnp.float32),
            jax.ShapeDtypeStruct((_B, _K), jnp.float32),
        ),
        mesh=mesh,
        compiler_params=pltpu.CompilerParams(
            needs_layout_passes=False, use_tc_tiling_on_sc=True),
        scratch_types=[
            pltpu.VMEM((8, _K), jnp.float32),
            pltpu.VMEM((8, _K), jnp.float32),
            pltpu.VMEM((8, _K), jnp.float32),
            pltpu.VMEM((_K,), jnp.int32),
            pltpu.VMEM((_K,), jnp.int32),
            pltpu.VMEM((_S, _K), jnp.float32),
            pltpu.VMEM((_P, _K), jnp.float32),
            pltpu.VMEM((_S, _K), jnp.float32),
            pltpu.VMEM((_P, _K), jnp.float32),
            pltpu.SemaphoreType.DMA,
            pltpu.SemaphoreType.DMA,
        ],
    )
    return f(c, p, state_t, param_t)


def kernel(state_particles, param_particles, log_weights):
    u = jax.random.uniform(jax.random.key(42), (_B, _K), dtype=jnp.float32)
    positions = (jnp.arange(_K, dtype=jnp.float32)[None, :] + u) / _K

    c = _prep(log_weights)
    new_s, new_p, nw = _sc_resample(
        c, positions,
        jnp.transpose(state_particles, (0, 2, 1)),
        jnp.transpose(param_particles, (0, 2, 1)),
    )
    new_log_weights = _finish(nw)
    return (jnp.transpose(new_s, (0, 2, 1)), jnp.transpose(new_p, (0, 2, 1)),
            new_log_weights)


# speedup vs baseline: 8.7008x; 1.5187x over previous
import functools

import jax
import jax.numpy as jnp
from jax import lax
from jax.experimental import pallas as pl
from jax.experimental.pallas import tpu as pltpu
from jax.experimental.pallas import tpu_sc as plsc

_B, _K, _S, _P = 1024, 1024, 32, 16
_NC, _NS = 2, 16
_NWORK = _NC * _NS
_RPW = _B // _NWORK
_L = 16
_NCH = _K // _L
_EPS = 1e-10


def _prep_body(lw_ref, c_ref):
    lw = lw_ref[...]
    w = jax.nn.softmax(lw, axis=-1)
    prop = 0.5 * w + jnp.float32(0.5 / _K)
    rows = prop.shape[0]
    lane = lax.broadcasted_iota(jnp.int32, (rows, _K), 1)
    inblk = lane & 127
    c = prop
    for i in range(1, 128):
        shifted = jnp.concatenate(
            [jnp.zeros((rows, 1), jnp.float32), c[:, : _K - 1]], axis=1)
        c = jnp.where(inblk == i, c + shifted, c)
    off = jnp.zeros((rows, _K), jnp.float32)
    running = jnp.zeros((rows, 1), jnp.float32)
    for b in range(1, _K // 128):
        s_prev = jnp.sum(
            jnp.where(lane == b * 128 - 1, c, 0.0), axis=-1, keepdims=True)
        running = running + s_prev
        off = jnp.where(lane >= b * 128, jnp.broadcast_to(running, off.shape),
                        off)
    c_ref[...] = c + off


def _prep(log_weights):
    blk = 128
    return pl.pallas_call(
        _prep_body,
        grid=(_B // blk,),
        in_specs=[pl.BlockSpec((blk, _K), lambda i: (i, 0))],
        out_specs=pl.BlockSpec((blk, _K), lambda i: (i, 0)),
        out_shape=jax.ShapeDtypeStruct((_B, _K), jnp.float32),
    )(log_weights)


def _fin_body(nw_ref, out_ref):
    l = jnp.log(nw_ref[...] + _EPS)
    mx = jnp.max(l, axis=-1, keepdims=True)
    lse = mx + jnp.log(jnp.sum(jnp.exp(l - mx), axis=-1, keepdims=True))
    out_ref[...] = l - lse


def _finish(nw):
    blk = 128
    return pl.pallas_call(
        _fin_body,
        grid=(_B // blk,),
        in_specs=[pl.BlockSpec((blk, _K), lambda i: (i, 0))],
        out_specs=pl.BlockSpec((blk, _K), lambda i: (i, 0)),
        out_shape=jax.ShapeDtypeStruct((_B, _K), jnp.float32),
    )(nw)


def _splat(x):
    return jnp.full((_L,), x, jnp.int32)


def _sc_body(c_hbm, p_hbm, state_hbm, param_hbm,
             out_s, out_p, out_nw,
             c_blk, p_blk, nw_blk, hist, idx_s,
             sslab, pslab, oslab_s, oslab_p,
             sem_in, sem_out):
    wid = lax.axis_index("s") * _NC + lax.axis_index("c")

    def tr_body(t, _):
        r0 = (wid * 4 + t) * 8
        pltpu.sync_copy(c_hbm.at[pl.ds(r0, 8)], c_blk)
        pltpu.sync_copy(p_hbm.at[pl.ds(r0, 8)], p_blk)

        def row_body(s, _):
            b = r0 + s
            in_s = pltpu.async_copy(state_hbm.at[b], sslab, sem_in)
            in_p = pltpu.async_copy(param_hbm.at[b], pslab, sem_in)

            def zero_body(i, _):
                hist[pl.ds(i * _L, _L)] = jnp.zeros((_L,), jnp.int32)
                return 0

            lax.fori_loop(0, _NCH, zero_body, 0)

            def m_body(j, _):
                c16 = c_blk[s, pl.ds(j * _L, _L)]
                t16 = c16 * jnp.float32(_K)
                i0 = t16.astype(jnp.int32)
                base = jnp.maximum(i0 - 1, 0)
                m = base
                for d in range(3):
                    cand = base + d
                    valid = cand < _K
                    pv = plsc.load_gather(
                        p_blk, [_splat(s), jnp.minimum(cand, _K - 1)],
                        mask=valid)
                    m = m + (valid & (pv <= c16)).astype(jnp.int32)
                plsc.addupdate_scatter(
                    hist, [jnp.minimum(m, _K - 1)],
                    jnp.full((_L,), 1, jnp.int32), mask=m < _K)
                return 0

            lax.fori_loop(0, _NCH, m_body, 0)

            def cs_body(j, carry):
                h16 = hist[pl.ds(j * _L, _L)]
                cs = plsc.cumsum(h16) + carry
                idx16 = jnp.minimum(cs, _K - 1)
                idx_s[pl.ds(j * _L, _L)] = idx16
                cg = plsc.load_gather(c_blk, [_splat(s), idx16])
                cgm = plsc.load_gather(
                    c_blk, [_splat(s), jnp.maximum(idx16 - 1, 0)])
                prop_g = cg - jnp.where(idx16 > 0, cgm, 0.0)
                w_g = 2.0 * prop_g - jnp.float32(1.0 / _K)
                nw_blk[s, pl.ds(j * _L, _L)] = w_g / (prop_g + _EPS)
                return jnp.max(cs)

            lax.fori_loop(0, _NCH, cs_body, jnp.int32(0))

            in_s.wait()
            in_p.wait()
            @pl.when(t * 8 + s > 0)
            def _():
                pltpu.make_async_copy(state_hbm.at[b], oslab_s, sem_out).wait()
                pltpu.make_async_copy(param_hbm.at[b], oslab_p, sem_out).wait()

            def g_body(j, _):
                sl = pl.ds(j * _L, _L)
                idx16 = idx_s[sl]
                for g0 in range(0, _S, 8):
                    vals = [plsc.load_gather(sslab, [_splat(ss), idx16])
                            for ss in range(g0, g0 + 8)]
                    for k, ss in enumerate(range(g0, g0 + 8)):
                        oslab_s[ss, sl] = vals[k]
                for g0 in range(0, _P, 8):
                    vals = [plsc.load_gather(pslab, [_splat(ss), idx16])
                            for ss in range(g0, g0 + 8)]
                    for k, ss in enumerate(range(g0, g0 + 8)):
                        oslab_p[ss, sl] = vals[k]
                return 0

            lax.fori_loop(0, _NCH, g_body, 0)

            pltpu.async_copy(oslab_s, out_s.at[b], sem_out)
            pltpu.async_copy(oslab_p, out_p.at[b], sem_out)
            return 0

        lax.fori_loop(0, 8, row_body, 0)
        pltpu.sync_copy(nw_blk, out_nw.at[pl.ds(r0, 8)])
        return 0

    lax.fori_loop(0, _RPW // 8, tr_body, 0)
    pltpu.make_async_copy(state_hbm.at[0], oslab_s, sem_out).wait()
    pltpu.make_async_copy(param_hbm.at[0], oslab_p, sem_out).wait()


def _sc_resample(c, p, state_t, param_t):
    mesh = plsc.VectorSubcoreMesh(core_axis_name="c", subcore_axis_name="s")
    f = pl.kernel(
        _sc_body,
        out_type=(
            jax.ShapeDtypeStruct((_B, _S, _K), jnp.float32),
            jax.ShapeDtypeStruct((_B, _P, _K), jnp.float32),
            jax.ShapeDtypeStruct((_B, _K), jnp.float32),
        ),
        mesh=mesh,
        compiler_params=pltpu.CompilerParams(
            needs_layout_passes=False, use_tc_tiling_on_sc=True),
        scratch_types=[
            pltpu.VMEM((8, _K), jnp.float32),
            pltpu.VMEM((8, _K), jnp.float32),
            pltpu.VMEM((8, _K), jnp.float32),
            pltpu.VMEM((_K,), jnp.int32),
            pltpu.VMEM((_K,), jnp.int32),
            pltpu.VMEM((_S, _K), jnp.float32),
            pltpu.VMEM((_P, _K), jnp.float32),
            pltpu.VMEM((_S, _K), jnp.float32),
            pltpu.VMEM((_P, _K), jnp.float32),
            pltpu.SemaphoreType.DMA,
            pltpu.SemaphoreType.DMA,
        ],
    )
    return f(c, p, state_t, param_t)


def kernel(state_particles, param_particles, log_weights):
    u = jax.random.uniform(jax.random.key(42), (_B, _K), dtype=jnp.float32)
    positions = (jnp.arange(_K, dtype=jnp.float32)[None, :] + u) / _K

    c = _prep(log_weights)
    new_s, new_p, nw = _sc_resample(
        c, positions,
        jnp.transpose(state_particles, (0, 2, 1)),
        jnp.transpose(param_particles, (0, 2, 1)),
    )
    new_log_weights = _finish(nw)
    return (jnp.transpose(new_s, (0, 2, 1)), jnp.transpose(new_p, (0, 2, 1)),
            new_log_weights)


# speedup vs baseline: 11.0359x; 1.2684x over previous
import functools

import jax
import jax.numpy as jnp
from jax import lax
from jax.experimental import pallas as pl
from jax.experimental.pallas import tpu as pltpu
from jax.experimental.pallas import tpu_sc as plsc

_B, _K, _S, _P = 1024, 1024, 32, 16
_NC, _NS = 2, 16
_NWORK = _NC * _NS
_RPW = _B // _NWORK
_L = 16
_NCH = _K // _L
_EPS = 1e-10


def _prep_body(lw_ref, prop_ref):
    lw = lw_ref[...]
    w = jax.nn.softmax(lw, axis=-1)
    prop_ref[...] = 0.5 * w + jnp.float32(0.5 / _K)


def _scan_body(y_ref, o_ref):
    wdt = y_ref.shape[1]
    o_ref[0, :] = y_ref[0, :]
    for p in range(1, 128):
        o_ref[p, :] = o_ref[p - 1, :] + y_ref[p, :]
    s = o_ref[127, :][None, :]
    lane = lax.broadcasted_iota(jnp.int32, (1, wdt), 1)
    g = lane & 7
    z1 = jnp.zeros((1, 1), jnp.float32)
    off = jnp.zeros((1, wdt), jnp.float32)
    for b in range(1, 8):
        off_sh = jnp.concatenate([z1, off[:, :-1]], axis=1)
        s_sh = jnp.concatenate([z1, s[:, :-1]], axis=1)
        off = jnp.where(g == b, off_sh + s_sh, off)
    o_ref[...] = o_ref[...] + off


def _prep(log_weights):
    blk = 128
    prop = pl.pallas_call(
        _prep_body,
        grid=(_B // blk,),
        in_specs=[pl.BlockSpec((blk, _K), lambda i: (i, 0))],
        out_specs=pl.BlockSpec((blk, _K), lambda i: (i, 0)),
        out_shape=jax.ShapeDtypeStruct((_B, _K), jnp.float32),
    )(log_weights)
    y = prop.reshape(_B, 8, 128).transpose(2, 0, 1).reshape(128, _B * 8)
    wblk = 512
    yc = pl.pallas_call(
        _scan_body,
        grid=(_B * 8 // wblk,),
        in_specs=[pl.BlockSpec((128, wblk), lambda i: (0, i))],
        out_specs=pl.BlockSpec((128, wblk), lambda i: (0, i)),
        out_shape=jax.ShapeDtypeStruct((128, _B * 8), jnp.float32),
    )(y)
    return yc.reshape(128, _B, 8).transpose(1, 2, 0).reshape(_B, _K)


def _fin_body(nw_ref, out_ref):
    l = jnp.log(nw_ref[...] + _EPS)
    mx = jnp.max(l, axis=-1, keepdims=True)
    lse = mx + jnp.log(jnp.sum(jnp.exp(l - mx), axis=-1, keepdims=True))
    out_ref[...] = l - lse


def _finish(nw):
    blk = 128
    return pl.pallas_call(
        _fin_body,
        grid=(_B // blk,),
        in_specs=[pl.BlockSpec((blk, _K), lambda i: (i, 0))],
        out_specs=pl.BlockSpec((blk, _K), lambda i: (i, 0)),
        out_shape=jax.ShapeDtypeStruct((_B, _K), jnp.float32),
    )(nw)


def _splat(x):
    return jnp.full((_L,), x, jnp.int32)


def _sc_body(c_hbm, p_hbm, state_hbm, param_hbm,
             out_s, out_p, out_nw,
             c_blk, p_blk, nw_blk, hist, idx_s,
             sslab, pslab, oslab_s, oslab_p,
             sem_in, sem_out):
    wid = lax.axis_index("s") * _NC + lax.axis_index("c")

    def tr_body(t, _):
        r0 = (wid * 4 + t) * 8
        pltpu.sync_copy(c_hbm.at[pl.ds(r0, 8)], c_blk)
        pltpu.sync_copy(p_hbm.at[pl.ds(r0, 8)], p_blk)

        def row_body(s, _):
            b = r0 + s
            in_s = pltpu.async_copy(state_hbm.at[b], sslab, sem_in)
            in_p = pltpu.async_copy(param_hbm.at[b], pslab, sem_in)

            def zero_body(i, _):
                hist[pl.ds(i * _L, _L)] = jnp.zeros((_L,), jnp.int32)
                return 0

            lax.fori_loop(0, _NCH, zero_body, 0)

            def m_body(j, _):
                c16 = c_blk[s, pl.ds(j * _L, _L)]
                t16 = c16 * jnp.float32(_K)
                i0 = t16.astype(jnp.int32)
                base = jnp.maximum(i0 - 1, 0)
                m = base
                for d in range(3):
                    cand = base + d
                    valid = cand < _K
                    pv = plsc.load_gather(
                        p_blk, [_splat(s), jnp.minimum(cand, _K - 1)],
                        mask=valid)
                    m = m + (valid & (pv <= c16)).astype(jnp.int32)
                plsc.addupdate_scatter(
                    hist, [jnp.minimum(m, _K - 1)],
                    jnp.full((_L,), 1, jnp.int32), mask=m < _K)
                return 0

            lax.fori_loop(0, _NCH, m_body, 0)

            def cs_body(j, carry):
                h16 = hist[pl.ds(j * _L, _L)]
                cs = plsc.cumsum(h16) + carry
                idx16 = jnp.minimum(cs, _K - 1)
                idx_s[pl.ds(j * _L, _L)] = idx16
                cg = plsc.load_gather(c_blk, [_splat(s), idx16])
                cgm = plsc.load_gather(
                    c_blk, [_splat(s), jnp.maximum(idx16 - 1, 0)])
                prop_g = cg - jnp.where(idx16 > 0, cgm, 0.0)
                w_g = 2.0 * prop_g - jnp.float32(1.0 / _K)
                nw_blk[s, pl.ds(j * _L, _L)] = w_g / (prop_g + _EPS)
                return jnp.max(cs)

            lax.fori_loop(0, _NCH, cs_body, jnp.int32(0))

            in_s.wait()
            in_p.wait()
            @pl.when(t * 8 + s > 0)
            def _():
                pltpu.make_async_copy(state_hbm.at[b], oslab_s, sem_out).wait()
                pltpu.make_async_copy(param_hbm.at[b], oslab_p, sem_out).wait()

            def g_body(j, _):
                sl = pl.ds(j * _L, _L)
                idx16 = idx_s[sl]
                for g0 in range(0, _S, 8):
                    vals = [plsc.load_gather(sslab, [_splat(ss), idx16])
                            for ss in range(g0, g0 + 8)]
                    for k, ss in enumerate(range(g0, g0 + 8)):
                        oslab_s[ss, sl] = vals[k]
                for g0 in range(0, _P, 8):
                    vals = [plsc.load_gather(pslab, [_splat(ss), idx16])
                            for ss in range(g0, g0 + 8)]
                    for k, ss in enumerate(range(g0, g0 + 8)):
                        oslab_p[ss, sl] = vals[k]
                return 0

            lax.fori_loop(0, _NCH, g_body, 0)

            pltpu.async_copy(oslab_s, out_s.at[b], sem_out)
            pltpu.async_copy(oslab_p, out_p.at[b], sem_out)
            return 0

        lax.fori_loop(0, 8, row_body, 0)
        pltpu.sync_copy(nw_blk, out_nw.at[pl.ds(r0, 8)])
        return 0

    lax.fori_loop(0, _RPW // 8, tr_body, 0)
    pltpu.make_async_copy(state_hbm.at[0], oslab_s, sem_out).wait()
    pltpu.make_async_copy(param_hbm.at[0], oslab_p, sem_out).wait()


def _sc_resample(c, p, state_t, param_t):
    mesh = plsc.VectorSubcoreMesh(core_axis_name="c", subcore_axis_name="s")
    f = pl.kernel(
        _sc_body,
        out_type=(
            jax.ShapeDtypeStruct((_B, _S, _K), jnp.float32),
            jax.ShapeDtypeStruct((_B, _P, _K), jnp.float32),
            jax.ShapeDtypeStruct((_B, _K), jnp.float32),
        ),
        mesh=mesh,
        compiler_params=pltpu.CompilerParams(
            needs_layout_passes=False, use_tc_tiling_on_sc=True),
        scratch_types=[
            pltpu.VMEM((8, _K), jnp.float32),
            pltpu.VMEM((8, _K), jnp.float32),
            pltpu.VMEM((8, _K), jnp.float32),
            pltpu.VMEM((_K,), jnp.int32),
            pltpu.VMEM((_K,), jnp.int32),
            pltpu.VMEM((_S, _K), jnp.float32),
            pltpu.VMEM((_P, _K), jnp.float32),
            pltpu.VMEM((_S, _K), jnp.float32),
            pltpu.VMEM((_P, _K), jnp.float32),
            pltpu.SemaphoreType.DMA,
            pltpu.SemaphoreType.DMA,
        ],
    )
    return f(c, p, state_t, param_t)


def kernel(state_particles, param_particles, log_weights):
    u = jax.random.uniform(jax.random.key(42), (_B, _K), dtype=jnp.float32)
    positions = (jnp.arange(_K, dtype=jnp.float32)[None, :] + u) / _K

    c = _prep(log_weights)
    new_s, new_p, nw = _sc_resample(
        c, positions,
        jnp.transpose(state_particles, (0, 2, 1)),
        jnp.transpose(param_particles, (0, 2, 1)),
    )
    new_log_weights = _finish(nw)
    return (jnp.transpose(new_s, (0, 2, 1)), jnp.transpose(new_p, (0, 2, 1)),
            new_log_weights)


# speedup vs baseline: 11.1155x; 1.0072x over previous
import functools

import jax
import jax.numpy as jnp
from jax import lax
from jax.experimental import pallas as pl
from jax.experimental.pallas import tpu as pltpu
from jax.experimental.pallas import tpu_sc as plsc

_B, _K, _S, _P = 1024, 1024, 32, 16
_NC, _NS = 2, 16
_NWORK = _NC * _NS
_RPW = _B // _NWORK
_L = 16
_NCH = _K // _L
_EPS = 1e-10


def _prep_body(lw_ref, prop_ref):
    lw = lw_ref[...]
    w = jax.nn.softmax(lw, axis=-1)
    prop_ref[...] = 0.5 * w + jnp.float32(0.5 / _K)


def _scan_body(y_ref, o_ref):
    wdt = y_ref.shape[1]
    o_ref[0, :] = y_ref[0, :]
    for p in range(1, 128):
        o_ref[p, :] = o_ref[p - 1, :] + y_ref[p, :]
    s = o_ref[127, :][None, :]
    lane = lax.broadcasted_iota(jnp.int32, (1, wdt), 1)
    g = lane & 7
    z1 = jnp.zeros((1, 1), jnp.float32)
    off = jnp.zeros((1, wdt), jnp.float32)
    for b in range(1, 8):
        off_sh = jnp.concatenate([z1, off[:, :-1]], axis=1)
        s_sh = jnp.concatenate([z1, s[:, :-1]], axis=1)
        off = jnp.where(g == b, off_sh + s_sh, off)
    o_ref[...] = o_ref[...] + off


def _prep(log_weights):
    blk = 128
    prop = pl.pallas_call(
        _prep_body,
        grid=(_B // blk,),
        in_specs=[pl.BlockSpec((blk, _K), lambda i: (i, 0))],
        out_specs=pl.BlockSpec((blk, _K), lambda i: (i, 0)),
        out_shape=jax.ShapeDtypeStruct((_B, _K), jnp.float32),
    )(log_weights)
    y = prop.reshape(_B, 8, 128).transpose(2, 0, 1).reshape(128, _B * 8)
    wblk = 512
    yc = pl.pallas_call(
        _scan_body,
        grid=(_B * 8 // wblk,),
        in_specs=[pl.BlockSpec((128, wblk), lambda i: (0, i))],
        out_specs=pl.BlockSpec((128, wblk), lambda i: (0, i)),
        out_shape=jax.ShapeDtypeStruct((128, _B * 8), jnp.float32),
    )(y)
    return yc.reshape(128, _B, 8).transpose(1, 2, 0).reshape(_B, _K)


def _fin_body(nw_ref, out_ref):
    l = jnp.log(nw_ref[...] + _EPS)
    mx = jnp.max(l, axis=-1, keepdims=True)
    lse = mx + jnp.log(jnp.sum(jnp.exp(l - mx), axis=-1, keepdims=True))
    out_ref[...] = l - lse


def _finish(nw):
    blk = 128
    return pl.pallas_call(
        _fin_body,
        grid=(_B // blk,),
        in_specs=[pl.BlockSpec((blk, _K), lambda i: (i, 0))],
        out_specs=pl.BlockSpec((blk, _K), lambda i: (i, 0)),
        out_shape=jax.ShapeDtypeStruct((_B, _K), jnp.float32),
    )(nw)


def _splat(x):
    return jnp.full((_L,), x, jnp.int32)


def _sc_body(c_hbm, p_hbm, state_hbm, param_hbm,
             out_s, out_p, out_nw,
             c_blk, p_blk, nw_blk, hist, idx_s,
             sslab, pslab, oslab_s, oslab_p,
             sem_in, sem_out):
    wid = lax.axis_index("s") * _NC + lax.axis_index("c")

    def zero_body(i, _):
        hist[pl.ds(i * _L, _L)] = jnp.zeros((_L,), jnp.int32)
        return 0

    lax.fori_loop(0, _NCH, zero_body, 0)

    def tr_body(t, _):
        r0 = (wid * 4 + t) * 8
        pltpu.sync_copy(c_hbm.at[pl.ds(r0, 8)], c_blk)
        pltpu.sync_copy(p_hbm.at[pl.ds(r0, 8)], p_blk)

        def row_body(s, _):
            b = r0 + s
            in_s = pltpu.async_copy(state_hbm.at[b], sslab, sem_in)
            in_p = pltpu.async_copy(param_hbm.at[b], pslab, sem_in)

            def m_body(j, _):
                c16 = c_blk[s, pl.ds(j * _L, _L)]
                t16 = c16 * jnp.float32(_K)
                i0 = t16.astype(jnp.int32)
                base = jnp.maximum(i0 - 1, 0)
                m = base
                for d in range(3):
                    cand = base + d
                    valid = cand < _K
                    pv = plsc.load_gather(
                        p_blk, [_splat(s), jnp.minimum(cand, _K - 1)],
                        mask=valid)
                    m = m + (valid & (pv <= c16)).astype(jnp.int32)
                plsc.addupdate_scatter(
                    hist, [jnp.minimum(m, _K - 1)],
                    jnp.full((_L,), 1, jnp.int32), mask=m < _K)
                return 0

            lax.fori_loop(0, _NCH, m_body, 0)

            def cs_body(j, carry):
                h16 = hist[pl.ds(j * _L, _L)]
                hist[pl.ds(j * _L, _L)] = jnp.zeros((_L,), jnp.int32)
                cs = plsc.cumsum(h16) + carry
                idx16 = jnp.minimum(cs, _K - 1)
                idx_s[pl.ds(j * _L, _L)] = idx16
                cg = plsc.load_gather(c_blk, [_splat(s), idx16])
                cgm = plsc.load_gather(
                    c_blk, [_splat(s), jnp.maximum(idx16 - 1, 0)])
                prop_g = cg - jnp.where(idx16 > 0, cgm, 0.0)
                w_g = 2.0 * prop_g - jnp.float32(1.0 / _K)
                nw_blk[s, pl.ds(j * _L, _L)] = w_g / (prop_g + _EPS)
                return jnp.max(cs)

            lax.fori_loop(0, _NCH, cs_body, jnp.int32(0))

            in_s.wait()
            in_p.wait()
            @pl.when(t * 8 + s > 0)
            def _():
                pltpu.make_async_copy(state_hbm.at[b], oslab_s, sem_out).wait()
                pltpu.make_async_copy(param_hbm.at[b], oslab_p, sem_out).wait()

            def g_body(j, _):
                sl = pl.ds(j * _L, _L)
                idx16 = idx_s[sl]
                for g0 in range(0, _S, 16):
                    vals = [plsc.load_gather(sslab, [_splat(ss), idx16])
                            for ss in range(g0, g0 + 16)]
                    for k, ss in enumerate(range(g0, g0 + 16)):
                        oslab_s[ss, sl] = vals[k]
                vals = [plsc.load_gather(pslab, [_splat(ss), idx16])
                        for ss in range(_P)]
                for ss in range(_P):
                    oslab_p[ss, sl] = vals[ss]
                return 0

            lax.fori_loop(0, _NCH, g_body, 0)

            pltpu.async_copy(oslab_s, out_s.at[b], sem_out)
            pltpu.async_copy(oslab_p, out_p.at[b], sem_out)
            return 0

        lax.fori_loop(0, 8, row_body, 0)
        pltpu.sync_copy(nw_blk, out_nw.at[pl.ds(r0, 8)])
        return 0

    lax.fori_loop(0, _RPW // 8, tr_body, 0)
    pltpu.make_async_copy(state_hbm.at[0], oslab_s, sem_out).wait()
    pltpu.make_async_copy(param_hbm.at[0], oslab_p, sem_out).wait()


def _sc_resample(c, p, state_t, param_t):
    mesh = plsc.VectorSubcoreMesh(core_axis_name="c", subcore_axis_name="s")
    f = pl.kernel(
        _sc_body,
        out_type=(
            jax.ShapeDtypeStruct((_B, _S, _K), jnp.float32),
            jax.ShapeDtypeStruct((_B, _P, _K), jnp.float32),
            jax.ShapeDtypeStruct((_B, _K), jnp.float32),
        ),
        mesh=mesh,
        compiler_params=pltpu.CompilerParams(
            needs_layout_passes=False, use_tc_tiling_on_sc=True),
        scratch_types=[
            pltpu.VMEM((8, _K), jnp.float32),
            pltpu.VMEM((8, _K), jnp.float32),
            pltpu.VMEM((8, _K), jnp.float32),
            pltpu.VMEM((_K,), jnp.int32),
            pltpu.VMEM((_K,), jnp.int32),
            pltpu.VMEM((_S, _K), jnp.float32),
            pltpu.VMEM((_P, _K), jnp.float32),
            pltpu.VMEM((_S, _K), jnp.float32),
            pltpu.VMEM((_P, _K), jnp.float32),
            pltpu.SemaphoreType.DMA,
            pltpu.SemaphoreType.DMA,
        ],
    )
    return f(c, p, state_t, param_t)


def kernel(state_particles, param_particles, log_weights):
    u = jax.random.uniform(jax.random.key(42), (_B, _K), dtype=jnp.float32)
    positions = (jnp.arange(_K, dtype=jnp.float32)[None, :] + u) / _K

    c = _prep(log_weights)
    new_s, new_p, nw = _sc_resample(
        c, positions,
        jnp.transpose(state_particles, (0, 2, 1)),
        jnp.transpose(param_particles, (0, 2, 1)),
    )
    new_log_weights = _finish(nw)
    return (jnp.transpose(new_s, (0, 2, 1)), jnp.transpose(new_p, (0, 2, 1)),
            new_log_weights)


# speedup vs baseline: 11.3386x; 1.0201x over previous
import functools

import jax
import jax.numpy as jnp
from jax import lax
from jax.experimental import pallas as pl
from jax.experimental.pallas import tpu as pltpu
from jax.experimental.pallas import tpu_sc as plsc

_B, _K, _S, _P = 1024, 1024, 32, 16
_NC, _NS = 2, 16
_NWORK = _NC * _NS
_RPW = _B // _NWORK
_L = 16
_NCH = _K // _L
_EPS = 1e-10


def _prep_body(lw_ref, prop_ref):
    lw = lw_ref[...]
    w = jax.nn.softmax(lw, axis=-1)
    prop_ref[...] = 0.5 * w + jnp.float32(0.5 / _K)


def _scan_body(y_ref, o_ref):
    wdt = y_ref.shape[1]
    o_ref[0, :] = y_ref[0, :]
    for p in range(1, 128):
        o_ref[p, :] = o_ref[p - 1, :] + y_ref[p, :]
    s = o_ref[127, :][None, :]
    lane = lax.broadcasted_iota(jnp.int32, (1, wdt), 1)
    g = lane & 7
    z1 = jnp.zeros((1, 1), jnp.float32)
    off = jnp.zeros((1, wdt), jnp.float32)
    for b in range(1, 8):
        off_sh = jnp.concatenate([z1, off[:, :-1]], axis=1)
        s_sh = jnp.concatenate([z1, s[:, :-1]], axis=1)
        off = jnp.where(g == b, off_sh + s_sh, off)
    o_ref[...] = o_ref[...] + off


def _prep(log_weights):
    blk = 128
    prop = pl.pallas_call(
        _prep_body,
        grid=(_B // blk,),
        in_specs=[pl.BlockSpec((blk, _K), lambda i: (i, 0))],
        out_specs=pl.BlockSpec((blk, _K), lambda i: (i, 0)),
        out_shape=jax.ShapeDtypeStruct((_B, _K), jnp.float32),
    )(log_weights)
    y = prop.reshape(_B, 8, 128).transpose(2, 0, 1).reshape(128, _B * 8)
    wblk = 512
    yc = pl.pallas_call(
        _scan_body,
        grid=(_B * 8 // wblk,),
        in_specs=[pl.BlockSpec((128, wblk), lambda i: (0, i))],
        out_specs=pl.BlockSpec((128, wblk), lambda i: (0, i)),
        out_shape=jax.ShapeDtypeStruct((128, _B * 8), jnp.float32),
    )(y)
    return yc.reshape(128, _B, 8).transpose(1, 2, 0).reshape(_B, _K)


def _fin_body(nw_ref, out_ref):
    l = jnp.log(nw_ref[...] + _EPS)
    mx = jnp.max(l, axis=-1, keepdims=True)
    lse = mx + jnp.log(jnp.sum(jnp.exp(l - mx), axis=-1, keepdims=True))
    out_ref[...] = l - lse


def _finish(nw):
    blk = 128
    return pl.pallas_call(
        _fin_body,
        grid=(_B // blk,),
        in_specs=[pl.BlockSpec((blk, _K), lambda i: (i, 0))],
        out_specs=pl.BlockSpec((blk, _K), lambda i: (i, 0)),
        out_shape=jax.ShapeDtypeStruct((_B, _K), jnp.float32),
    )(nw)


def _splat(x):
    return jnp.full((_L,), x, jnp.int32)


def _sc_body(c_hbm, p_hbm, state_hbm, param_hbm,
             out_s, out_p, out_nw,
             c_blk, p_blk, nw_blk, hist, idx_s,
             sslab, pslab, oslab_s, oslab_p,
             sem_in, sem_out):
    wid = lax.axis_index("s") * _NC + lax.axis_index("c")

    def zero_body(i, _):
        hist[pl.ds(i * _L, _L)] = jnp.zeros((_L,), jnp.int32)
        return 0

    lax.fori_loop(0, _NCH, zero_body, 0)

    def tr_body(t, _):
        r0 = (wid * 4 + t) * 8
        pltpu.sync_copy(c_hbm.at[pl.ds(r0, 8)], c_blk)
        pltpu.sync_copy(p_hbm.at[pl.ds(r0, 8)], p_blk)

        def row_body(s, _):
            b = r0 + s
            in_s = pltpu.async_copy(state_hbm.at[b], sslab, sem_in)
            in_p = pltpu.async_copy(param_hbm.at[b], pslab, sem_in)

            def m_one(j):
                c16 = c_blk[s, pl.ds(j * _L, _L)]
                t16 = c16 * jnp.float32(_K)
                i0 = t16.astype(jnp.int32)
                base = jnp.maximum(i0 - 1, 0)
                m = base
                for d in range(3):
                    cand = base + d
                    valid = cand < _K
                    pv = plsc.load_gather(
                        p_blk, [_splat(s), jnp.minimum(cand, _K - 1)],
                        mask=valid)
                    m = m + (valid & (pv <= c16)).astype(jnp.int32)
                plsc.addupdate_scatter(
                    hist, [jnp.minimum(m, _K - 1)],
                    jnp.full((_L,), 1, jnp.int32), mask=m < _K)

            def m_body(j, _):
                m_one(2 * j)
                m_one(2 * j + 1)
                return 0

            lax.fori_loop(0, _NCH // 2, m_body, 0)

            def cs_one(j, carry, h16):
                hist[pl.ds(j * _L, _L)] = jnp.zeros((_L,), jnp.int32)
                cs = plsc.cumsum(h16) + carry
                idx16 = jnp.minimum(cs, _K - 1)
                idx_s[pl.ds(j * _L, _L)] = idx16
                cg = plsc.load_gather(c_blk, [_splat(s), idx16])
                cgm = plsc.load_gather(
                    c_blk, [_splat(s), jnp.maximum(idx16 - 1, 0)])
                prop_g = cg - jnp.where(idx16 > 0, cgm, 0.0)
                w_g = 2.0 * prop_g - jnp.float32(1.0 / _K)
                nw_blk[s, pl.ds(j * _L, _L)] = w_g / (prop_g + _EPS)
                return jnp.max(cs)

            def cs_body(j, carry):
                h_a = hist[pl.ds((2 * j) * _L, _L)]
                h_b = hist[pl.ds((2 * j + 1) * _L, _L)]
                carry = cs_one(2 * j, carry, h_a)
                carry = cs_one(2 * j + 1, carry, h_b)
                return carry

            lax.fori_loop(0, _NCH // 2, cs_body, jnp.int32(0))

            in_s.wait()
            in_p.wait()
            @pl.when(t * 8 + s > 0)
            def _():
                pltpu.make_async_copy(state_hbm.at[b], oslab_s, sem_out).wait()
                pltpu.make_async_copy(param_hbm.at[b], oslab_p, sem_out).wait()

            def g_body(j, _):
                sl = pl.ds(j * _L, _L)
                idx16 = idx_s[sl]
                for g0 in range(0, _S, 16):
                    vals = [plsc.load_gather(sslab, [_splat(ss), idx16])
                            for ss in range(g0, g0 + 16)]
                    for k, ss in enumerate(range(g0, g0 + 16)):
                        oslab_s[ss, sl] = vals[k]
                vals = [plsc.load_gather(pslab, [_splat(ss), idx16])
                        for ss in range(_P)]
                for ss in range(_P):
                    oslab_p[ss, sl] = vals[ss]
                return 0

            lax.fori_loop(0, _NCH, g_body, 0)

            pltpu.async_copy(oslab_s, out_s.at[b], sem_out)
            pltpu.async_copy(oslab_p, out_p.at[b], sem_out)
            return 0

        lax.fori_loop(0, 8, row_body, 0)
        pltpu.sync_copy(nw_blk, out_nw.at[pl.ds(r0, 8)])
        return 0

    lax.fori_loop(0, _RPW // 8, tr_body, 0)
    pltpu.make_async_copy(state_hbm.at[0], oslab_s, sem_out).wait()
    pltpu.make_async_copy(param_hbm.at[0], oslab_p, sem_out).wait()


def _sc_resample(c, p, state_t, param_t):
    mesh = plsc.VectorSubcoreMesh(core_axis_name="c", subcore_axis_name="s")
    f = pl.kernel(
        _sc_body,
        out_type=(
            jax.ShapeDtypeStruct((_B, _S, _K), jnp.float32),
            jax.ShapeDtypeStruct((_B, _P, _K), jnp.float32),
            jax.ShapeDtypeStruct((_B, _K), jnp.float32),
        ),
        mesh=mesh,
        compiler_params=pltpu.CompilerParams(
            needs_layout_passes=False, use_tc_tiling_on_sc=True),
        scratch_types=[
            pltpu.VMEM((8, _K), jnp.float32),
            pltpu.VMEM((8, _K), jnp.float32),
            pltpu.VMEM((8, _K), jnp.float32),
            pltpu.VMEM((_K,), jnp.int32),
            pltpu.VMEM((_K,), jnp.int32),
            pltpu.VMEM((_S, _K), jnp.float32),
            pltpu.VMEM((_P, _K), jnp.float32),
            pltpu.VMEM((_S, _K), jnp.float32),
            pltpu.VMEM((_P, _K), jnp.float32),
            pltpu.SemaphoreType.DMA,
            pltpu.SemaphoreType.DMA,
        ],
    )
    return f(c, p, state_t, param_t)


def kernel(state_particles, param_particles, log_weights):
    u = jax.random.uniform(jax.random.key(42), (_B, _K), dtype=jnp.float32)
    positions = (jnp.arange(_K, dtype=jnp.float32)[None, :] + u) / _K

    c = _prep(log_weights)
    new_s, new_p, nw = _sc_resample(
        c, positions,
        jnp.transpose(state_particles, (0, 2, 1)),
        jnp.transpose(param_particles, (0, 2, 1)),
    )
    new_log_weights = _finish(nw)
    return (jnp.transpose(new_s, (0, 2, 1)), jnp.transpose(new_p, (0, 2, 1)),
            new_log_weights)
